# baseline (speedup 1.0000x reference)
import functools

import jax
import jax.numpy as jnp
from jax import lax
from jax.experimental import pallas as pl
from jax.experimental.pallas import tpu as pltpu
from jax.experimental.pallas import tpu_sc as plsc

_NC = 2
_NS = 16
_NW = _NC * _NS
_L = 16
_NBUF = 3


def _sc_gather_kernel(n_slabs, d, T, pe_hbm, x_hbm, out_hbm, x_v,
                      ia0, ia1, ia2, rows0, rows1, rows2,
                      g0, g1, g2, o0, o1, o2):
    per_w = n_slabs // _NW
    n_chunks = per_w // 2
    idxa = [ia0, ia1, ia2]
    rows = [rows0, rows1, rows2]
    gsem = [g0, g1, g2]
    osem = [o0, o1, o2]

    wid = lax.axis_index("s") * _NC + lax.axis_index("c")
    base_s = wid * per_w

    pltpu.sync_copy(x_hbm.at[pl.ds(base_s * T, per_w * T)], x_v)

    def build(c, b):
        a0 = (base_s + 2 * c) % 128
        for j in range(2 * T // _L):
            aval = a0 + (j * _L) // T
            idxa[b][pl.ds(j * _L, _L)] = jnp.full((_L,), aval, jnp.int32)

    def g_start(c, b):
        for k in range(2):
            pltpu.async_copy(
                pe_hbm.at[idxa[b].at[pl.ds(k * T, T)]],
                rows[b].at[k, pl.ds(0, T), pl.ds(0, d)], gsem[b])
            pltpu.async_copy(
                pe_hbm.at[x_v.at[pl.ds((2 * c + k) * T, T)]],
                rows[b].at[k, pl.ds(0, T), pl.ds(d, d)], gsem[b])

    def g_wait(c, b):
        for k in range(2):
            pltpu.make_async_copy(
                pe_hbm.at[idxa[b].at[pl.ds(k * T, T)]],
                rows[b].at[k, pl.ds(0, T), pl.ds(0, d)], gsem[b]).wait()
            pltpu.make_async_copy(
                pe_hbm.at[x_v.at[pl.ds((2 * c + k) * T, T)]],
                rows[b].at[k, pl.ds(0, T), pl.ds(d, d)], gsem[b]).wait()

    def _o_desc(c, b):
        return pltpu.make_async_copy(
            rows[b], out_hbm.at[pl.ds(base_s + 2 * c, 2)], osem[b])

    def o_start(c, b):
        _o_desc(c, b).start()

    def o_wait(c, b):
        _o_desc(c, b).wait()

    build(0, 0)
    g_start(0, 0)

    def loop_body(r, carry):
        for b in range(_NBUF):
            c = _NBUF * r + b
            bn = (b + 1) % _NBUF
            @pl.when(c >= 2)
            def _():
                o_wait(c - 2, bn)
            build(c + 1, bn)
            g_start(c + 1, bn)
            g_wait(c, b)
            o_start(c, b)
        return carry

    n_main = (n_chunks - 2) // _NBUF
    lax.fori_loop(0, n_main, loop_body, 0)

    c = n_chunks - 2
    b, bn = c % _NBUF, (c + 1) % _NBUF
    o_wait(c - 2, bn)
    build(c + 1, bn)
    g_start(c + 1, bn)
    g_wait(c, b)
    o_start(c, b)
    c = n_chunks - 1
    b = c % _NBUF
    o_wait(c - 2, (c + 1) % _NBUF)
    g_wait(c, b)
    o_start(c, b)
    o_wait(n_chunks - 2, (n_chunks - 2) % _NBUF)
    o_wait(n_chunks - 1, (n_chunks - 1) % _NBUF)


def kernel(x, pe):
    B, A, T = x.shape
    d = pe.shape[2]
    n_slabs = B * A
    per_w = n_slabs // _NW
    assert n_slabs % _NW == 0 and per_w % 2 == 0
    assert (per_w // 2 - 2) % _NBUF == 0 and per_w // 2 >= 5
    assert T % _L == 0 and d % _L == 0 and A == 128

    mesh = plsc.VectorSubcoreMesh(core_axis_name="c", subcore_axis_name="s")
    body = functools.partial(_sc_gather_kernel, n_slabs, d, T)
    run = pl.kernel(
        body,
        mesh=mesh,
        compiler_params=pltpu.CompilerParams(needs_layout_passes=False),
        out_type=jax.ShapeDtypeStruct((n_slabs, T, 2 * d), jnp.float32),
        scratch_types=(
            [pltpu.VMEM((per_w * T,), jnp.int32)]
            + [pltpu.VMEM((2 * T,), jnp.int32) for _ in range(_NBUF)]
            + [pltpu.VMEM((2, T, 2 * d), jnp.float32) for _ in range(_NBUF)]
            + [pltpu.SemaphoreType.DMA] * (2 * _NBUF)
        ),
    )
    out = run(pe.reshape(pe.shape[0], d), x.reshape(-1))
    return out.reshape(B, A, T, 2 * d)


# speedup vs baseline: 3.5910x; 3.5910x over previous
import functools

import jax
import jax.numpy as jnp
from jax import lax
from jax.experimental import pallas as pl
from jax.experimental.pallas import tpu as pltpu
from jax.experimental.pallas import tpu_sc as plsc

_NC = 2
_NS = 16
_NW = _NC * _NS
_L = 16
_NBUF = 3


def _sc_gather_kernel(n_slabs, d, T, A, pe_hbm, x_hbm, out_hbm, x_v, atoms_v,
                      rows0, rows1, rows2, g0, g1, g2, o0, o1, o2):
    per_w = n_slabs // _NW
    n_chunks = per_w // 2
    rows = [rows0, rows1, rows2]
    gsem = [g0, g1, g2]
    osem = [o0, o1, o2]

    wid = lax.axis_index("s") * _NC + lax.axis_index("c")
    base_s = wid * per_w

    pltpu.sync_copy(x_hbm.at[pl.ds(base_s * T, per_w * T)], x_v)
    a_lo = base_s % A
    pltpu.sync_copy(pe_hbm.at[pl.ds(a_lo, per_w)], atoms_v)

    def g_start(c, b):
        for k in range(2):
            pltpu.async_copy(
                pe_hbm.at[x_v.at[pl.ds((2 * c + k) * T, T)]],
                rows[b].at[k, pl.ds(0, T), pl.ds(d, d)], gsem[b])

    def g_wait(c, b):
        for k in range(2):
            pltpu.make_async_copy(
                pe_hbm.at[x_v.at[pl.ds((2 * c + k) * T, T)]],
                rows[b].at[k, pl.ds(0, T), pl.ds(d, d)], gsem[b]).wait()

    def repl(c, b):
        vals = [[atoms_v[2 * c + k, pl.ds(j * _L, _L)]
                 for j in range(d // _L)] for k in range(2)]

        def t_body(t, carry):
            for k in range(2):
                for j in range(d // _L):
                    rows[b][k, t, pl.ds(j * _L, _L)] = vals[k][j]
            return carry

        lax.fori_loop(0, T, t_body, 0)

    def _o_desc(c, b):
        return pltpu.make_async_copy(
            rows[b], out_hbm.at[pl.ds(base_s + 2 * c, 2)], osem[b])

    def o_start(c, b):
        _o_desc(c, b).start()

    def o_wait(c, b):
        _o_desc(c, b).wait()

    g_start(0, 0)
    repl(0, 0)

    def loop_body(r, carry):
        for b in range(_NBUF):
            c = _NBUF * r + b
            bn = (b + 1) % _NBUF
            @pl.when(c >= 2)
            def _():
                o_wait(c - 2, bn)
            g_start(c + 1, bn)
            repl(c + 1, bn)
            g_wait(c, b)
            o_start(c, b)
        return carry

    n_main = (n_chunks - 2) // _NBUF
    lax.fori_loop(0, n_main, loop_body, 0)

    c = n_chunks - 2
    b, bn = c % _NBUF, (c + 1) % _NBUF
    o_wait(c - 2, bn)
    g_start(c + 1, bn)
    repl(c + 1, bn)
    g_wait(c, b)
    o_start(c, b)
    c = n_chunks - 1
    b = c % _NBUF
    o_wait(c - 2, (c + 1) % _NBUF)
    g_wait(c, b)
    o_start(c, b)
    o_wait(n_chunks - 2, (n_chunks - 2) % _NBUF)
    o_wait(n_chunks - 1, (n_chunks - 1) % _NBUF)


def kernel(x, pe):
    B, A, T = x.shape
    d = pe.shape[2]
    n_slabs = B * A
    per_w = n_slabs // _NW
    assert n_slabs % _NW == 0 and per_w % 2 == 0
    assert (per_w // 2 - 2) % _NBUF == 0 and per_w // 2 >= 5
    assert T % _L == 0 and d % _L == 0 and A % per_w == 0

    mesh = plsc.VectorSubcoreMesh(core_axis_name="c", subcore_axis_name="s")
    body = functools.partial(_sc_gather_kernel, n_slabs, d, T, A)
    run = pl.kernel(
        body,
        mesh=mesh,
        compiler_params=pltpu.CompilerParams(needs_layout_passes=False),
        out_type=jax.ShapeDtypeStruct((n_slabs, T, 2 * d), jnp.float32),
        scratch_types=(
            [pltpu.VMEM((per_w * T,), jnp.int32)]
            + [pltpu.VMEM((per_w, d), jnp.float32)]
            + [pltpu.VMEM((2, T, 2 * d), jnp.float32) for _ in range(_NBUF)]
            + [pltpu.SemaphoreType.DMA] * (2 * _NBUF)
        ),
    )
    out = run(pe.reshape(pe.shape[0], d), x.reshape(-1))
    return out.reshape(B, A, T, 2 * d)
